# Initial kernel scaffold; baseline (speedup 1.0000x reference)
import jax
import jax.numpy as jnp
from jax.experimental import pallas as pl


def kernel(x, edge_index, W1l, b1, W1r, W2l, b2, W2r):
    raise NotImplementedError("write your pallas kernel here")


# speedup vs baseline: 3.5485x; 3.5485x over previous
import functools

import jax
import jax.numpy as jnp
from jax import lax
from jax.experimental import pallas as pl
from jax.experimental.pallas import tpu as pltpu
from jax.experimental.pallas import tpu_sc as plsc

N = 10000
E = 320000
D_IN = 128
D_HID = 512
D_OUT = 512

NC = 2
NS = 16
E_TILE = E // (NC * NS)
B = 80
NB = E_TILE // B
N_PAD = 10240
ROWS_T = N_PAD // NS


def _sc_segment_sum(src, dst, chunks, zf, zc, with_count):
  K = len(chunks)
  mesh = plsc.VectorSubcoreMesh(core_axis_name="c", subcore_axis_name="s")

  out_type = [jax.ShapeDtypeStruct((NC, K, N_PAD, D_IN), jnp.float32)]
  if with_count:
    out_type.append(jax.ShapeDtypeStruct((NC * N_PAD,), jnp.float32))

  scratch = [
      pltpu.VMEM((2, B), jnp.int32),
      pltpu.VMEM((2, B), jnp.int32),
      pltpu.VMEM((B, D_IN), jnp.float32),
      pltpu.VMEM((B,), jnp.float32),
      pltpu.VMEM_SHARED((N_PAD, D_IN), jnp.float32),
      pltpu.VMEM_SHARED((N_PAD,), jnp.float32),
      pltpu.SemaphoreType.DMA,
  ]

  @functools.partial(pl.kernel, mesh=mesh, out_type=out_type,
                     scratch_types=scratch)
  def k(*refs):
    (src_h, dst_h, *rest) = refs
    chunk_h = rest[:K]
    rest = rest[K:]
    zf_h, zc_h = rest[0], rest[1]
    rest = rest[2:]
    if with_count:
      feat_o, cnt_o = rest[0], rest[1]
      rest = rest[2:]
    else:
      feat_o = rest[0]
      cnt_o = None
      rest = rest[1:]
    src_v, dst_v, rows_v, ones_v, acc_s, accc_s, sem = rest

    c = lax.axis_index("c")
    s = lax.axis_index("s")
    row0 = s * ROWS_T
    base = (c * NS + s) * E_TILE

    if with_count:
      for i in range(B // 16):
        ones_v[pl.ds(i * 16, 16)] = jnp.ones((16,), jnp.float32)

    for kk in range(K):
      pltpu.sync_copy(zf_h, acc_s.at[pl.ds(row0, ROWS_T)])
      if with_count and kk == 0:
        pltpu.sync_copy(zc_h, accc_s.at[pl.ds(row0, ROWS_T)])
      plsc.subcore_barrier()

      def body(j, _):
        off = base + j * B
        pltpu.sync_copy(src_h.at[pl.ds(off, B)], src_v.at[0])
        pltpu.sync_copy(dst_h.at[pl.ds(off, B)], dst_v.at[0])
        pltpu.async_copy(chunk_h[kk].at[src_v.at[0]], rows_v, sem).wait()
        pltpu.sync_copy(rows_v, acc_s.at[dst_v.at[0]], add=True)
        if with_count and kk == 0:
          pltpu.sync_copy(ones_v, accc_s.at[dst_v.at[0]], add=True)
        return ()

      lax.fori_loop(0, NB, body, (), unroll=False)
      plsc.subcore_barrier()

      pltpu.sync_copy(acc_s.at[pl.ds(row0, ROWS_T)],
                      feat_o.at[c, kk, pl.ds(row0, ROWS_T)])
      if with_count and kk == 0:
        pltpu.sync_copy(accc_s.at[pl.ds(row0, ROWS_T)],
                        cnt_o.at[pl.ds(c * N_PAD + row0, ROWS_T)])

  outs = k(src, dst, *chunks, zf, zc)
  if with_count:
    return outs[0], outs[1]
  return outs[0], None


M_TILE = 400


def _tc1_body(x_ref, pf_ref, pc_ref, w1l_ref, b1_ref, w1r_ref,
              h0_ref, h1_ref, h2_ref, h3_ref):
  cnt = pc_ref[0, :, 0] + pc_ref[1, :, 0]
  inv = 1.0 / jnp.maximum(cnt, 1.0)
  agg = (pf_ref[0, 0] + pf_ref[1, 0]) * inv[:, None]
  z = lax.dot_general(agg, w1l_ref[...], (((1,), (1,)), ((), ())),
                      preferred_element_type=jnp.float32)
  z = z + b1_ref[...]
  z = z + lax.dot_general(x_ref[...], w1r_ref[...], (((1,), (1,)), ((), ())),
                          preferred_element_type=jnp.float32)
  h = jnp.maximum(z, 0.0)
  h0_ref[...] = h[:, 0:128]
  h1_ref[...] = h[:, 128:256]
  h2_ref[...] = h[:, 256:384]
  h3_ref[...] = h[:, 384:512]


def _tc2_body(h0_ref, h1_ref, h2_ref, h3_ref, p2_ref, pc_ref,
              w2l_ref, b2_ref, w2r_ref, out_ref):
  cnt = pc_ref[0, :, 0] + pc_ref[1, :, 0]
  inv = 1.0 / jnp.maximum(cnt, 1.0)
  psum = p2_ref[0] + p2_ref[1]
  agg = jnp.concatenate([psum[0], psum[1], psum[2], psum[3]], axis=1)
  agg = agg * inv[:, None]
  z = lax.dot_general(agg, w2l_ref[...], (((1,), (1,)), ((), ())),
                      preferred_element_type=jnp.float32)
  z = z + b2_ref[...]
  h = jnp.concatenate([h0_ref[...], h1_ref[...], h2_ref[...], h3_ref[...]],
                      axis=1)
  z = z + lax.dot_general(h, w2r_ref[...], (((1,), (1,)), ((), ())),
                          preferred_element_type=jnp.float32)
  out_ref[...] = jnp.maximum(z, 0.0)


def _chunk_spec():
  return pl.BlockSpec((M_TILE, 128), lambda i: (i, 0))


def kernel(x, edge_index, W1l, b1, W1r, W2l, b2, W2r):
  src = edge_index[0].astype(jnp.int32)
  dst = edge_index[1].astype(jnp.int32)

  zf = jnp.zeros((ROWS_T, D_IN), jnp.float32)
  zc = jnp.zeros((ROWS_T,), jnp.float32)

  p1, pc = _sc_segment_sum(src, dst, (x,), zf, zc, with_count=True)
  pc2 = pc.reshape(NC, N_PAD, 1)

  grid = N // M_TILE
  b1_2d = b1.reshape(1, D_HID)
  h_chunks = pl.pallas_call(
      _tc1_body,
      grid=(grid,),
      in_specs=[
          pl.BlockSpec((M_TILE, D_IN), lambda i: (i, 0)),
          pl.BlockSpec((NC, 1, M_TILE, D_IN), lambda i: (0, 0, i, 0)),
          pl.BlockSpec((NC, M_TILE, 1), lambda i: (0, i, 0)),
          pl.BlockSpec((D_HID, D_IN), lambda i: (0, 0)),
          pl.BlockSpec((1, D_HID), lambda i: (0, 0)),
          pl.BlockSpec((D_HID, D_IN), lambda i: (0, 0)),
      ],
      out_specs=[_chunk_spec(), _chunk_spec(), _chunk_spec(), _chunk_spec()],
      out_shape=[jax.ShapeDtypeStruct((N, 128), jnp.float32)] * 4,
  )(x, p1, pc2, W1l, b1_2d, W1r)

  p2, _ = _sc_segment_sum(src, dst, tuple(h_chunks), zf, zc,
                          with_count=False)

  b2_2d = b2.reshape(1, D_OUT)
  out = pl.pallas_call(
      _tc2_body,
      grid=(grid,),
      in_specs=[
          _chunk_spec(), _chunk_spec(), _chunk_spec(), _chunk_spec(),
          pl.BlockSpec((NC, 4, M_TILE, 128), lambda i: (0, 0, i, 0)),
          pl.BlockSpec((NC, M_TILE, 1), lambda i: (0, i, 0)),
          pl.BlockSpec((D_OUT, D_HID), lambda i: (0, 0)),
          pl.BlockSpec((1, D_OUT), lambda i: (0, 0)),
          pl.BlockSpec((D_OUT, D_HID), lambda i: (0, 0)),
      ],
      out_specs=pl.BlockSpec((M_TILE, D_OUT), lambda i: (i, 0)),
      out_shape=jax.ShapeDtypeStruct((N, D_OUT), jnp.float32),
  )(*h_chunks, p2, pc2, W2l, b2_2d, W2r)
  return out


# speedup vs baseline: 5.7259x; 1.6136x over previous
import functools

import jax
import jax.numpy as jnp
from jax import lax
from jax.experimental import pallas as pl
from jax.experimental.pallas import tpu as pltpu
from jax.experimental.pallas import tpu_sc as plsc

N = 10000
E = 320000
D_IN = 128
D_HID = 512
D_OUT = 512

NC = 2
NS = 16
E_TILE = E // (NC * NS)
B = 80
NB = E_TILE // B
N_PAD = 10240
ROWS_T = N_PAD // NS


def _sc_segment_sum(src, dst, chunks, zf, zc, with_count):
  K = len(chunks)
  mesh = plsc.VectorSubcoreMesh(core_axis_name="c", subcore_axis_name="s")

  out_type = [jax.ShapeDtypeStruct((NC, K, N_PAD, D_IN), jnp.float32)]
  if with_count:
    out_type.append(jax.ShapeDtypeStruct((NC * N_PAD,), jnp.float32))

  scratch = [
      pltpu.VMEM((2, B), jnp.int32),
      pltpu.VMEM((2, B), jnp.int32),
      pltpu.VMEM((2, B, D_IN), jnp.float32),
      pltpu.VMEM((B,), jnp.float32),
      pltpu.VMEM_SHARED((N_PAD, D_IN), jnp.float32),
      pltpu.VMEM_SHARED((N_PAD,), jnp.float32),
      pltpu.SemaphoreType.DMA,
      pltpu.SemaphoreType.DMA,
  ]

  @functools.partial(pl.kernel, mesh=mesh, out_type=out_type,
                     scratch_types=scratch)
  def k(*refs):
    (src_h, dst_h, *rest) = refs
    chunk_h = rest[:K]
    rest = rest[K:]
    zf_h, zc_h = rest[0], rest[1]
    rest = rest[2:]
    if with_count:
      feat_o, cnt_o = rest[0], rest[1]
      rest = rest[2:]
    else:
      feat_o = rest[0]
      cnt_o = None
      rest = rest[1:]
    src_v, dst_v, rows_v, ones_v, acc_s, accc_s, sem0, sem1 = rest

    c = lax.axis_index("c")
    s = lax.axis_index("s")
    row0 = s * ROWS_T
    base = (c * NS + s) * E_TILE

    if with_count:
      for i in range(B // 16):
        ones_v[pl.ds(i * 16, 16)] = jnp.ones((16,), jnp.float32)

    def load_idx(j, p):
      pltpu.sync_copy(src_h.at[pl.ds(base + j * B, B)], src_v.at[p])
      pltpu.sync_copy(dst_h.at[pl.ds(base + j * B, B)], dst_v.at[p])

    for kk in range(K):
      ch = chunk_h[kk]
      count_now = with_count and kk == 0
      pltpu.sync_copy(zf_h, acc_s.at[pl.ds(row0, ROWS_T)])
      if count_now:
        pltpu.sync_copy(zc_h, accc_s.at[pl.ds(row0, ROWS_T)])
      plsc.subcore_barrier()

      def scat(p):
        pltpu.sync_copy(rows_v.at[p], acc_s.at[dst_v.at[p]], add=True)
        if count_now:
          pltpu.sync_copy(ones_v, accc_s.at[dst_v.at[p]], add=True)

      load_idx(0, 0)
      pltpu.async_copy(ch.at[src_v.at[0]], rows_v.at[0], sem0)

      def body(jj, _):
        j = 2 * jj
        load_idx(j + 1, 1)
        pltpu.async_copy(ch.at[src_v.at[1]], rows_v.at[1], sem1)
        pltpu.make_async_copy(ch.at[src_v.at[0]], rows_v.at[0], sem0).wait()
        scat(0)

        @pl.when(j + 2 < NB)
        def _():
          load_idx(j + 2, 0)
          pltpu.async_copy(ch.at[src_v.at[0]], rows_v.at[0], sem0)

        pltpu.make_async_copy(ch.at[src_v.at[1]], rows_v.at[1], sem1).wait()
        scat(1)
        return ()

      lax.fori_loop(0, NB // 2, body, (), unroll=False)
      pltpu.make_async_copy(ch.at[src_v.at[0]], rows_v.at[0], sem0).wait()
      scat(0)
      plsc.subcore_barrier()

      pltpu.sync_copy(acc_s.at[pl.ds(row0, ROWS_T)],
                      feat_o.at[c, kk, pl.ds(row0, ROWS_T)])
      if count_now:
        pltpu.sync_copy(accc_s.at[pl.ds(row0, ROWS_T)],
                        cnt_o.at[pl.ds(c * N_PAD + row0, ROWS_T)])

  outs = k(src, dst, *chunks, zf, zc)
  if with_count:
    return outs[0], outs[1]
  return outs[0], None


M_TILE = 400


def _tc1_body(x_ref, pf_ref, pc_ref, w1l_ref, b1_ref, w1r_ref,
              h0_ref, h1_ref, h2_ref, h3_ref):
  cnt = pc_ref[0, :, 0] + pc_ref[1, :, 0]
  inv = 1.0 / jnp.maximum(cnt, 1.0)
  agg = (pf_ref[0, 0] + pf_ref[1, 0]) * inv[:, None]
  z = lax.dot_general(agg, w1l_ref[...], (((1,), (1,)), ((), ())),
                      preferred_element_type=jnp.float32)
  z = z + b1_ref[...]
  z = z + lax.dot_general(x_ref[...], w1r_ref[...], (((1,), (1,)), ((), ())),
                          preferred_element_type=jnp.float32)
  h = jnp.maximum(z, 0.0)
  h0_ref[...] = h[:, 0:128]
  h1_ref[...] = h[:, 128:256]
  h2_ref[...] = h[:, 256:384]
  h3_ref[...] = h[:, 384:512]


def _tc2_body(h0_ref, h1_ref, h2_ref, h3_ref, p2_ref, pc_ref,
              w2l_ref, b2_ref, w2r_ref, out_ref):
  cnt = pc_ref[0, :, 0] + pc_ref[1, :, 0]
  inv = 1.0 / jnp.maximum(cnt, 1.0)
  psum = p2_ref[0] + p2_ref[1]
  agg = jnp.concatenate([psum[0], psum[1], psum[2], psum[3]], axis=1)
  agg = agg * inv[:, None]
  z = lax.dot_general(agg, w2l_ref[...], (((1,), (1,)), ((), ())),
                      preferred_element_type=jnp.float32)
  z = z + b2_ref[...]
  h = jnp.concatenate([h0_ref[...], h1_ref[...], h2_ref[...], h3_ref[...]],
                      axis=1)
  z = z + lax.dot_general(h, w2r_ref[...], (((1,), (1,)), ((), ())),
                          preferred_element_type=jnp.float32)
  out_ref[...] = jnp.maximum(z, 0.0)


def _chunk_spec():
  return pl.BlockSpec((M_TILE, 128), lambda i: (i, 0))


def kernel(x, edge_index, W1l, b1, W1r, W2l, b2, W2r):
  src = edge_index[0].astype(jnp.int32)
  dst = edge_index[1].astype(jnp.int32)

  zf = jnp.zeros((ROWS_T, D_IN), jnp.float32)
  zc = jnp.zeros((ROWS_T,), jnp.float32)

  p1, pc = _sc_segment_sum(src, dst, (x,), zf, zc, with_count=True)
  pc2 = pc.reshape(NC, N_PAD, 1)

  grid = N // M_TILE
  b1_2d = b1.reshape(1, D_HID)
  h_chunks = pl.pallas_call(
      _tc1_body,
      grid=(grid,),
      in_specs=[
          pl.BlockSpec((M_TILE, D_IN), lambda i: (i, 0)),
          pl.BlockSpec((NC, 1, M_TILE, D_IN), lambda i: (0, 0, i, 0)),
          pl.BlockSpec((NC, M_TILE, 1), lambda i: (0, i, 0)),
          pl.BlockSpec((D_HID, D_IN), lambda i: (0, 0)),
          pl.BlockSpec((1, D_HID), lambda i: (0, 0)),
          pl.BlockSpec((D_HID, D_IN), lambda i: (0, 0)),
      ],
      out_specs=[_chunk_spec(), _chunk_spec(), _chunk_spec(), _chunk_spec()],
      out_shape=[jax.ShapeDtypeStruct((N, 128), jnp.float32)] * 4,
  )(x, p1, pc2, W1l, b1_2d, W1r)

  p2, _ = _sc_segment_sum(src, dst, tuple(h_chunks), zf, zc,
                          with_count=False)

  b2_2d = b2.reshape(1, D_OUT)
  out = pl.pallas_call(
      _tc2_body,
      grid=(grid,),
      in_specs=[
          _chunk_spec(), _chunk_spec(), _chunk_spec(), _chunk_spec(),
          pl.BlockSpec((NC, 4, M_TILE, 128), lambda i: (0, 0, i, 0)),
          pl.BlockSpec((NC, M_TILE, 1), lambda i: (0, i, 0)),
          pl.BlockSpec((D_OUT, D_HID), lambda i: (0, 0)),
          pl.BlockSpec((1, D_OUT), lambda i: (0, 0)),
          pl.BlockSpec((D_OUT, D_HID), lambda i: (0, 0)),
      ],
      out_specs=pl.BlockSpec((M_TILE, D_OUT), lambda i: (i, 0)),
      out_shape=jax.ShapeDtypeStruct((N, D_OUT), jnp.float32),
  )(*h_chunks, p2, pc2, W2l, b2_2d, W2r)
  return out


# speedup vs baseline: 6.9955x; 1.2217x over previous
import functools

import jax
import jax.numpy as jnp
from jax import lax
from jax.experimental import pallas as pl
from jax.experimental.pallas import tpu as pltpu
from jax.experimental.pallas import tpu_sc as plsc

N = 10000
E = 320000
D_IN = 128
D_HID = 512
D_OUT = 512

NC = 2
NS = 16
E_TILE = E // (NC * NS)
B = 80
NB = E_TILE // B
N_PAD = 10240
ROWS_T = N_PAD // NS


def _sc_segment_sum(src, dst, chunks, zf, zc, with_count):
  K = len(chunks)
  mesh = plsc.VectorSubcoreMesh(core_axis_name="c", subcore_axis_name="s")

  out_type = [jax.ShapeDtypeStruct((NC, K, N_PAD, D_IN), jnp.float32)]
  if with_count:
    out_type.append(jax.ShapeDtypeStruct((NC * N_PAD,), jnp.float32))

  scratch = [
      pltpu.VMEM((E_TILE,), jnp.int32),
      pltpu.VMEM((2, B), jnp.int32),
      pltpu.VMEM((2, B, D_IN), jnp.float32),
      pltpu.VMEM((B,), jnp.float32),
      pltpu.VMEM_SHARED((N_PAD, D_IN), jnp.float32),
      pltpu.VMEM_SHARED((N_PAD,), jnp.float32),
      pltpu.SemaphoreType.DMA,
      pltpu.SemaphoreType.DMA,
      pltpu.SemaphoreType.DMA,
      pltpu.SemaphoreType.DMA,
      pltpu.SemaphoreType.DMA,
      pltpu.SemaphoreType.DMA,
  ]

  @functools.partial(pl.kernel, mesh=mesh, out_type=out_type,
                     scratch_types=scratch)
  def k(*refs):
    (src_h, dst_h, *rest) = refs
    chunk_h = rest[:K]
    rest = rest[K:]
    zf_h, zc_h = rest[0], rest[1]
    rest = rest[2:]
    if with_count:
      feat_o, cnt_o = rest[0], rest[1]
      rest = rest[2:]
    else:
      feat_o = rest[0]
      cnt_o = None
      rest = rest[1:]
    (src_v, dst_v, rows_v, ones_v, acc_s, accc_s,
     g0, g1, s0, s1, c0, c1) = rest

    c = lax.axis_index("c")
    s = lax.axis_index("s")
    row0 = s * ROWS_T
    base = (c * NS + s) * E_TILE

    pltpu.sync_copy(src_h.at[pl.ds(base, E_TILE)], src_v)

    if with_count:
      for i in range(B // 16):
        ones_v[pl.ds(i * 16, 16)] = jnp.ones((16,), jnp.float32)

    def ld_dst(j, p):
      pltpu.sync_copy(dst_h.at[pl.ds(base + j * B, B)], dst_v.at[p])

    for kk in range(K):
      ch = chunk_h[kk]
      count_now = with_count and kk == 0
      gsem = (g0, g1)
      ssem = (s0, s1)
      csem = (c0, c1)

      def gather(j, p):
        pltpu.async_copy(ch.at[src_v.at[pl.ds(j * B, B)]], rows_v.at[p],
                         gsem[p])

      def gwait(j, p):
        pltpu.make_async_copy(ch.at[src_v.at[pl.ds(j * B, B)]],
                              rows_v.at[p], gsem[p]).wait()

      def scat(p):
        pltpu.async_copy(rows_v.at[p], acc_s.at[dst_v.at[p]], ssem[p],
                         add=True)
        if count_now:
          pltpu.async_copy(ones_v, accc_s.at[dst_v.at[p]], csem[p],
                           add=True)

      def swait(p):
        pltpu.make_async_copy(rows_v.at[p], acc_s.at[dst_v.at[p]],
                              ssem[p]).wait()
        if count_now:
          pltpu.make_async_copy(ones_v, accc_s.at[dst_v.at[p]],
                                csem[p]).wait()

      pltpu.sync_copy(zf_h, acc_s.at[pl.ds(row0, ROWS_T)])
      if count_now:
        pltpu.sync_copy(zc_h, accc_s.at[pl.ds(row0, ROWS_T)])
      plsc.subcore_barrier()

      ld_dst(0, 0)
      gather(0, 0)
      ld_dst(1, 1)
      gather(1, 1)

      def body(jj, _):
        j = 2 * jj
        gwait(j, 0)
        scat(0)
        gwait(j + 1, 1)
        scat(1)

        @pl.when(j + 2 < NB)
        def _():
          swait(0)
          ld_dst(j + 2, 0)
          gather(j + 2, 0)

        @pl.when(j + 3 < NB)
        def _():
          swait(1)
          ld_dst(j + 3, 1)
          gather(j + 3, 1)
        return ()

      lax.fori_loop(0, NB // 2, body, (), unroll=False)
      gwait(NB - 1, 0)
      scat(0)
      swait(0)
      swait(1)
      plsc.subcore_barrier()

      pltpu.sync_copy(acc_s.at[pl.ds(row0, ROWS_T)],
                      feat_o.at[c, kk, pl.ds(row0, ROWS_T)])
      if count_now:
        pltpu.sync_copy(accc_s.at[pl.ds(row0, ROWS_T)],
                        cnt_o.at[pl.ds(c * N_PAD + row0, ROWS_T)])

  outs = k(src, dst, *chunks, zf, zc)
  if with_count:
    return outs[0], outs[1]
  return outs[0], None


M_TILE = 400


def _tc1_body(x_ref, pf_ref, pc_ref, w1l_ref, b1_ref, w1r_ref,
              h0_ref, h1_ref, h2_ref, h3_ref):
  cnt = pc_ref[0, :, 0] + pc_ref[1, :, 0]
  inv = 1.0 / jnp.maximum(cnt, 1.0)
  agg = (pf_ref[0, 0] + pf_ref[1, 0]) * inv[:, None]
  z = lax.dot_general(agg, w1l_ref[...], (((1,), (1,)), ((), ())),
                      preferred_element_type=jnp.float32)
  z = z + b1_ref[...]
  z = z + lax.dot_general(x_ref[...], w1r_ref[...], (((1,), (1,)), ((), ())),
                          preferred_element_type=jnp.float32)
  h = jnp.maximum(z, 0.0)
  h0_ref[...] = h[:, 0:128]
  h1_ref[...] = h[:, 128:256]
  h2_ref[...] = h[:, 256:384]
  h3_ref[...] = h[:, 384:512]


def _tc2_body(h0_ref, h1_ref, h2_ref, h3_ref, p2_ref, pc_ref,
              w2l_ref, b2_ref, w2r_ref, out_ref):
  cnt = pc_ref[0, :, 0] + pc_ref[1, :, 0]
  inv = 1.0 / jnp.maximum(cnt, 1.0)
  psum = p2_ref[0] + p2_ref[1]
  agg = jnp.concatenate([psum[0], psum[1], psum[2], psum[3]], axis=1)
  agg = agg * inv[:, None]
  z = lax.dot_general(agg, w2l_ref[...], (((1,), (1,)), ((), ())),
                      preferred_element_type=jnp.float32)
  z = z + b2_ref[...]
  h = jnp.concatenate([h0_ref[...], h1_ref[...], h2_ref[...], h3_ref[...]],
                      axis=1)
  z = z + lax.dot_general(h, w2r_ref[...], (((1,), (1,)), ((), ())),
                          preferred_element_type=jnp.float32)
  out_ref[...] = jnp.maximum(z, 0.0)


def _chunk_spec():
  return pl.BlockSpec((M_TILE, 128), lambda i: (i, 0))


def kernel(x, edge_index, W1l, b1, W1r, W2l, b2, W2r):
  src = edge_index[0].astype(jnp.int32)
  dst = edge_index[1].astype(jnp.int32)

  zf = jnp.zeros((ROWS_T, D_IN), jnp.float32)
  zc = jnp.zeros((ROWS_T,), jnp.float32)

  p1, pc = _sc_segment_sum(src, dst, (x,), zf, zc, with_count=True)
  pc2 = pc.reshape(NC, N_PAD, 1)

  grid = N // M_TILE
  b1_2d = b1.reshape(1, D_HID)
  h_chunks = pl.pallas_call(
      _tc1_body,
      grid=(grid,),
      in_specs=[
          pl.BlockSpec((M_TILE, D_IN), lambda i: (i, 0)),
          pl.BlockSpec((NC, 1, M_TILE, D_IN), lambda i: (0, 0, i, 0)),
          pl.BlockSpec((NC, M_TILE, 1), lambda i: (0, i, 0)),
          pl.BlockSpec((D_HID, D_IN), lambda i: (0, 0)),
          pl.BlockSpec((1, D_HID), lambda i: (0, 0)),
          pl.BlockSpec((D_HID, D_IN), lambda i: (0, 0)),
      ],
      out_specs=[_chunk_spec(), _chunk_spec(), _chunk_spec(), _chunk_spec()],
      out_shape=[jax.ShapeDtypeStruct((N, 128), jnp.float32)] * 4,
  )(x, p1, pc2, W1l, b1_2d, W1r)

  p2, _ = _sc_segment_sum(src, dst, tuple(h_chunks), zf, zc,
                          with_count=False)

  b2_2d = b2.reshape(1, D_OUT)
  out = pl.pallas_call(
      _tc2_body,
      grid=(grid,),
      in_specs=[
          _chunk_spec(), _chunk_spec(), _chunk_spec(), _chunk_spec(),
          pl.BlockSpec((NC, 4, M_TILE, 128), lambda i: (0, 0, i, 0)),
          pl.BlockSpec((NC, M_TILE, 1), lambda i: (0, i, 0)),
          pl.BlockSpec((D_OUT, D_HID), lambda i: (0, 0)),
          pl.BlockSpec((1, D_OUT), lambda i: (0, 0)),
          pl.BlockSpec((D_OUT, D_HID), lambda i: (0, 0)),
      ],
      out_specs=pl.BlockSpec((M_TILE, D_OUT), lambda i: (i, 0)),
      out_shape=jax.ShapeDtypeStruct((N, D_OUT), jnp.float32),
  )(*h_chunks, p2, pc2, W2l, b2_2d, W2r)
  return out


# speedup vs baseline: 8.2906x; 1.1851x over previous
import functools

import jax
import jax.numpy as jnp
from jax import lax
from jax.experimental import pallas as pl
from jax.experimental.pallas import tpu as pltpu
from jax.experimental.pallas import tpu_sc as plsc

N = 10000
E = 320000
D_IN = 128
D_HID = 512
D_OUT = 512

NC = 2
NS = 16
E_TILE = E // (NC * NS)
B = 80
NB = E_TILE // B
N_PAD = 10112
ROWS_T = N_PAD // NS
N_PADC = 10240
ROWS_C = N_PADC // NS


def _sc_segment_sum(src, dst, chunks, zf, zc, with_count):
  K = len(chunks)
  mesh = plsc.VectorSubcoreMesh(core_axis_name="c", subcore_axis_name="s")

  out_type = [jax.ShapeDtypeStruct((NC, K, N_PAD, D_IN), jnp.float32)]
  if with_count:
    out_type.append(jax.ShapeDtypeStruct((NC * N_PADC,), jnp.float32))

  scratch = [
      pltpu.VMEM((E_TILE,), jnp.int32),
      pltpu.VMEM((3, B), jnp.int32),
      pltpu.VMEM((3, B, D_IN), jnp.float32),
      pltpu.VMEM((B,), jnp.float32),
      pltpu.VMEM_SHARED((N_PAD, D_IN), jnp.float32),
  ] + ([pltpu.VMEM_SHARED((N_PADC,), jnp.float32)] if with_count else []) \
    + [pltpu.SemaphoreType.DMA] * (9 if with_count else 6)

  @functools.partial(pl.kernel, mesh=mesh, out_type=out_type,
                     scratch_types=scratch)
  def k(*refs):
    (src_h, dst_h, *rest) = refs
    chunk_h = rest[:K]
    rest = rest[K:]
    zf_h, zc_h = rest[0], rest[1]
    rest = rest[2:]
    if with_count:
      feat_o, cnt_o = rest[0], rest[1]
      rest = rest[2:]
    else:
      feat_o = rest[0]
      cnt_o = None
      rest = rest[1:]
    src_v, dst_v, rows_v, ones_v, acc_s = rest[:5]
    rest = rest[5:]
    if with_count:
      accc_s = rest[0]
      rest = rest[1:]
    else:
      accc_s = None
    gsem = rest[0:3]
    ssem = rest[3:6]
    csem = rest[6:9] if with_count else None

    c = lax.axis_index("c")
    s = lax.axis_index("s")
    row0 = s * ROWS_T
    base = (c * NS + s) * E_TILE

    pltpu.sync_copy(src_h.at[pl.ds(base, E_TILE)], src_v)

    if with_count:
      for i in range(B // 16):
        ones_v[pl.ds(i * 16, 16)] = jnp.ones((16,), jnp.float32)

    def ld_dst(j, p):
      pltpu.sync_copy(dst_h.at[pl.ds(base + j * B, B)], dst_v.at[p])

    for kk in range(K):
      ch = chunk_h[kk]
      count_now = with_count and kk == 0

      def gather(j, p):
        pltpu.async_copy(ch.at[src_v.at[pl.ds(j * B, B)]], rows_v.at[p],
                         gsem[p])

      def gwait(j, p):
        pltpu.make_async_copy(ch.at[src_v.at[pl.ds(j * B, B)]],
                              rows_v.at[p], gsem[p]).wait()

      def scat(p):
        pltpu.async_copy(rows_v.at[p], acc_s.at[dst_v.at[p]], ssem[p],
                         add=True)
        if count_now:
          pltpu.async_copy(ones_v, accc_s.at[dst_v.at[p]], csem[p],
                           add=True)

      def swait(p):
        pltpu.make_async_copy(rows_v.at[p], acc_s.at[dst_v.at[p]],
                              ssem[p]).wait()
        if count_now:
          pltpu.make_async_copy(ones_v, accc_s.at[dst_v.at[p]],
                                csem[p]).wait()

      def step(j, p, do_swait, guard=True):
        gwait(j, p)
        scat(p)
        p2 = (p + 2) % 3

        def prefetch():
          if do_swait:
            swait(p2)
          ld_dst(j + 2, p2)
          gather(j + 2, p2)

        if guard:
          pl.when(j + 2 < NB)(prefetch)
        else:
          prefetch()

      pltpu.sync_copy(zf_h, acc_s.at[pl.ds(row0, ROWS_T)])
      if count_now:
        pltpu.sync_copy(zc_h, accc_s.at[pl.ds(s * ROWS_C, ROWS_C)])
      plsc.subcore_barrier()

      ld_dst(0, 0)
      gather(0, 0)
      ld_dst(1, 1)
      gather(1, 1)
      step(0, 0, False, guard=False)

      def body(jj, _):
        j = 3 * jj + 1
        step(j, 1, True)
        step(j + 1, 2, True)
        step(j + 2, 0, True)
        return ()

      lax.fori_loop(0, (NB - 2) // 3, body, (), unroll=False)
      gwait(NB - 1, 1)
      scat(1)
      for p in (2, 0, 1):
        swait(p)
      plsc.subcore_barrier()

      pltpu.sync_copy(acc_s.at[pl.ds(row0, ROWS_T)],
                      feat_o.at[c, kk, pl.ds(row0, ROWS_T)])
      if count_now:
        pltpu.sync_copy(accc_s.at[pl.ds(s * ROWS_C, ROWS_C)],
                        cnt_o.at[pl.ds(c * N_PADC + s * ROWS_C, ROWS_C)])

  outs = k(src, dst, *chunks, zf, zc)
  if with_count:
    return outs[0], outs[1]
  return outs[0], None


M_TILE = 400


def _tc1_body(x_ref, pf_ref, pc_ref, w1l_ref, b1_ref, w1r_ref,
              h0_ref, h1_ref, h2_ref, h3_ref):
  cnt = pc_ref[0, :, 0] + pc_ref[1, :, 0]
  inv = 1.0 / jnp.maximum(cnt, 1.0)
  agg = (pf_ref[0, 0] + pf_ref[1, 0]) * inv[:, None]
  z = lax.dot_general(agg, w1l_ref[...], (((1,), (1,)), ((), ())),
                      preferred_element_type=jnp.float32)
  z = z + b1_ref[...]
  z = z + lax.dot_general(x_ref[...], w1r_ref[...], (((1,), (1,)), ((), ())),
                          preferred_element_type=jnp.float32)
  h = jnp.maximum(z, 0.0)
  h0_ref[...] = h[:, 0:128]
  h1_ref[...] = h[:, 128:256]
  h2_ref[...] = h[:, 256:384]
  h3_ref[...] = h[:, 384:512]


def _tc2_body(h0_ref, h1_ref, h2_ref, h3_ref, p2_ref, pc_ref,
              w2l_ref, b2_ref, w2r_ref, out_ref):
  cnt = pc_ref[0, :, 0] + pc_ref[1, :, 0]
  inv = 1.0 / jnp.maximum(cnt, 1.0)
  psum = p2_ref[0] + p2_ref[1]
  agg = jnp.concatenate([psum[0], psum[1], psum[2], psum[3]], axis=1)
  agg = agg * inv[:, None]
  z = lax.dot_general(agg, w2l_ref[...], (((1,), (1,)), ((), ())),
                      preferred_element_type=jnp.float32)
  z = z + b2_ref[...]
  h = jnp.concatenate([h0_ref[...], h1_ref[...], h2_ref[...], h3_ref[...]],
                      axis=1)
  z = z + lax.dot_general(h, w2r_ref[...], (((1,), (1,)), ((), ())),
                          preferred_element_type=jnp.float32)
  out_ref[...] = jnp.maximum(z, 0.0)


def _chunk_spec():
  return pl.BlockSpec((M_TILE, 128), lambda i: (i, 0))


def kernel(x, edge_index, W1l, b1, W1r, W2l, b2, W2r):
  src = edge_index[0].astype(jnp.int32)
  dst = edge_index[1].astype(jnp.int32)

  zf = jnp.zeros((ROWS_T, D_IN), jnp.float32)
  zc = jnp.zeros((ROWS_C,), jnp.float32)

  p1, pc = _sc_segment_sum(src, dst, (x,), zf, zc, with_count=True)
  pc2 = pc.reshape(NC, N_PADC, 1)

  grid = N // M_TILE
  b1_2d = b1.reshape(1, D_HID)
  h_chunks = pl.pallas_call(
      _tc1_body,
      grid=(grid,),
      in_specs=[
          pl.BlockSpec((M_TILE, D_IN), lambda i: (i, 0)),
          pl.BlockSpec((NC, 1, M_TILE, D_IN), lambda i: (0, 0, i, 0)),
          pl.BlockSpec((NC, M_TILE, 1), lambda i: (0, i, 0)),
          pl.BlockSpec((D_HID, D_IN), lambda i: (0, 0)),
          pl.BlockSpec((1, D_HID), lambda i: (0, 0)),
          pl.BlockSpec((D_HID, D_IN), lambda i: (0, 0)),
      ],
      out_specs=[_chunk_spec(), _chunk_spec(), _chunk_spec(), _chunk_spec()],
      out_shape=[jax.ShapeDtypeStruct((N, 128), jnp.float32)] * 4,
  )(x, p1, pc2, W1l, b1_2d, W1r)

  p2, _ = _sc_segment_sum(src, dst, tuple(h_chunks), zf, zc,
                          with_count=False)

  b2_2d = b2.reshape(1, D_OUT)
  out = pl.pallas_call(
      _tc2_body,
      grid=(grid,),
      in_specs=[
          _chunk_spec(), _chunk_spec(), _chunk_spec(), _chunk_spec(),
          pl.BlockSpec((NC, 4, M_TILE, 128), lambda i: (0, 0, i, 0)),
          pl.BlockSpec((NC, M_TILE, 1), lambda i: (0, i, 0)),
          pl.BlockSpec((D_OUT, D_HID), lambda i: (0, 0)),
          pl.BlockSpec((1, D_OUT), lambda i: (0, 0)),
          pl.BlockSpec((D_OUT, D_HID), lambda i: (0, 0)),
      ],
      out_specs=pl.BlockSpec((M_TILE, D_OUT), lambda i: (i, 0)),
      out_shape=jax.ShapeDtypeStruct((N, D_OUT), jnp.float32),
  )(*h_chunks, p2, pc2, W2l, b2_2d, W2r)
  return out


# speedup vs baseline: 9.8592x; 1.1892x over previous
import functools

import jax
import jax.numpy as jnp
from jax import lax
from jax.experimental import pallas as pl
from jax.experimental.pallas import tpu as pltpu
from jax.experimental.pallas import tpu_sc as plsc

N = 10000
E = 320000
D_IN = 128
D_HID = 512
D_OUT = 512

NC = 2
NS = 16
E_TILE = E // (NC * NS)
B = 80
NB = E_TILE // B
N_PAD = 10112
ROWS_T = N_PAD // NS
N_PADC = 10240
ROWS_C = N_PADC // NS


def _sc_segment_sum(src, dst, chunks, zf, zc, with_count):
  K = len(chunks)
  mesh = plsc.VectorSubcoreMesh(core_axis_name="c", subcore_axis_name="s")

  out_type = [jax.ShapeDtypeStruct((NC, K, N_PAD, D_IN), jnp.float32)]
  if with_count:
    out_type.append(jax.ShapeDtypeStruct((NC * N_PADC,), jnp.float32))

  scratch = [
      pltpu.VMEM((E_TILE,), jnp.int32),
      pltpu.VMEM((3, B), jnp.int32),
      pltpu.VMEM((3, B, D_IN), jnp.float32),
      pltpu.VMEM((B,), jnp.float32),
      pltpu.VMEM_SHARED((N_PAD, D_IN), jnp.float32),
  ] + ([pltpu.VMEM_SHARED((N_PADC,), jnp.float32)] if with_count else []) \
    + [pltpu.SemaphoreType.DMA] * (12 if with_count else 9)

  @functools.partial(pl.kernel, mesh=mesh, out_type=out_type,
                     scratch_types=scratch)
  def k(*refs):
    (src_h, dst_h, *rest) = refs
    chunk_h = rest[:K]
    rest = rest[K:]
    zf_h, zc_h = rest[0], rest[1]
    rest = rest[2:]
    if with_count:
      feat_o, cnt_o = rest[0], rest[1]
      rest = rest[2:]
    else:
      feat_o = rest[0]
      cnt_o = None
      rest = rest[1:]
    src_v, dst_v, rows_v, ones_v, acc_s = rest[:5]
    rest = rest[5:]
    if with_count:
      accc_s = rest[0]
      rest = rest[1:]
    else:
      accc_s = None
    gsem = rest[0:3]
    ssem = rest[3:6]
    dsem = rest[6:9]
    csem = rest[9:12] if with_count else None

    c = lax.axis_index("c")
    s = lax.axis_index("s")
    row0 = s * ROWS_T
    base = (c * NS + s) * E_TILE

    pltpu.sync_copy(src_h.at[pl.ds(base, E_TILE)], src_v)

    if with_count:
      for i in range(B // 16):
        ones_v[pl.ds(i * 16, 16)] = jnp.ones((16,), jnp.float32)

    def ld_dst(j, p):
      pltpu.async_copy(dst_h.at[pl.ds(base + j * B, B)], dst_v.at[p],
                       dsem[p])

    def ld_dst_wait(j, p):
      pltpu.make_async_copy(dst_h.at[pl.ds(base + j * B, B)], dst_v.at[p],
                            dsem[p]).wait()

    for kk in range(K):
      ch = chunk_h[kk]
      count_now = with_count and kk == 0

      def gather(j, p):
        pltpu.async_copy(ch.at[src_v.at[pl.ds(j * B, B)]], rows_v.at[p],
                         gsem[p])

      def gwait(j, p):
        pltpu.make_async_copy(ch.at[src_v.at[pl.ds(j * B, B)]],
                              rows_v.at[p], gsem[p]).wait()

      def scat(p):
        pltpu.async_copy(rows_v.at[p], acc_s.at[dst_v.at[p]], ssem[p],
                         add=True)
        if count_now:
          pltpu.async_copy(ones_v, accc_s.at[dst_v.at[p]], csem[p],
                           add=True)

      def swait(p):
        pltpu.make_async_copy(rows_v.at[p], acc_s.at[dst_v.at[p]],
                              ssem[p]).wait()
        if count_now:
          pltpu.make_async_copy(ones_v, accc_s.at[dst_v.at[p]],
                                csem[p]).wait()

      def step(j, p, do_swait, guard=True):
        gwait(j, p)
        ld_dst_wait(j, p)
        scat(p)
        p2 = (p + 2) % 3

        def prefetch():
          if do_swait:
            swait(p2)
          ld_dst(j + 2, p2)
          gather(j + 2, p2)

        if guard:
          pl.when(j + 2 < NB)(prefetch)
        else:
          prefetch()

      ld_dst(0, 0)
      gather(0, 0)
      ld_dst(1, 1)
      gather(1, 1)
      pltpu.sync_copy(zf_h, acc_s.at[pl.ds(row0, ROWS_T)])
      if count_now:
        pltpu.sync_copy(zc_h, accc_s.at[pl.ds(s * ROWS_C, ROWS_C)])
      plsc.subcore_barrier()

      step(0, 0, False, guard=False)

      def body(jj, _):
        j = 3 * jj + 1
        step(j, 1, True)
        step(j + 1, 2, True)
        step(j + 2, 0, True)
        return ()

      lax.fori_loop(0, (NB - 2) // 3, body, (), unroll=False)
      gwait(NB - 1, 1)
      ld_dst_wait(NB - 1, 1)
      scat(1)
      for p in (2, 0, 1):
        swait(p)
      plsc.subcore_barrier()

      pltpu.sync_copy(acc_s.at[pl.ds(row0, ROWS_T)],
                      feat_o.at[c, kk, pl.ds(row0, ROWS_T)])
      if count_now:
        pltpu.sync_copy(accc_s.at[pl.ds(s * ROWS_C, ROWS_C)],
                        cnt_o.at[pl.ds(c * N_PADC + s * ROWS_C, ROWS_C)])

  outs = k(src, dst, *chunks, zf, zc)
  if with_count:
    return outs[0], outs[1]
  return outs[0], None


M_TILE = 400


def _tc1_body(x_ref, pf_ref, pc_ref, w1l_ref, b1_ref, w1r_ref,
              h0_ref, h1_ref, h2_ref, h3_ref):
  cnt = pc_ref[0, :, 0] + pc_ref[1, :, 0]
  inv = 1.0 / jnp.maximum(cnt, 1.0)
  agg = (pf_ref[0, 0] + pf_ref[1, 0]) * inv[:, None]
  z = lax.dot_general(agg, w1l_ref[...], (((1,), (1,)), ((), ())),
                      preferred_element_type=jnp.float32)
  z = z + b1_ref[...]
  z = z + lax.dot_general(x_ref[...], w1r_ref[...], (((1,), (1,)), ((), ())),
                          preferred_element_type=jnp.float32)
  h = jnp.maximum(z, 0.0)
  h0_ref[...] = h[:, 0:128]
  h1_ref[...] = h[:, 128:256]
  h2_ref[...] = h[:, 256:384]
  h3_ref[...] = h[:, 384:512]


def _tc2_body(h0_ref, h1_ref, h2_ref, h3_ref, p2_ref, pc_ref,
              w2l_ref, b2_ref, w2r_ref, out_ref):
  cnt = pc_ref[0, :, 0] + pc_ref[1, :, 0]
  inv = 1.0 / jnp.maximum(cnt, 1.0)
  psum = p2_ref[0] + p2_ref[1]
  agg = jnp.concatenate([psum[0], psum[1], psum[2], psum[3]], axis=1)
  agg = agg * inv[:, None]
  z = lax.dot_general(agg, w2l_ref[...], (((1,), (1,)), ((), ())),
                      preferred_element_type=jnp.float32)
  z = z + b2_ref[...]
  h = jnp.concatenate([h0_ref[...], h1_ref[...], h2_ref[...], h3_ref[...]],
                      axis=1)
  z = z + lax.dot_general(h, w2r_ref[...], (((1,), (1,)), ((), ())),
                          preferred_element_type=jnp.float32)
  out_ref[...] = jnp.maximum(z, 0.0)


def _chunk_spec():
  return pl.BlockSpec((M_TILE, 128), lambda i: (i, 0))


def kernel(x, edge_index, W1l, b1, W1r, W2l, b2, W2r):
  src = edge_index[0].astype(jnp.int32)
  dst = edge_index[1].astype(jnp.int32)

  zf = jnp.zeros((ROWS_T, D_IN), jnp.float32)
  zc = jnp.zeros((ROWS_C,), jnp.float32)

  p1, pc = _sc_segment_sum(src, dst, (x,), zf, zc, with_count=True)
  pc2 = pc.reshape(NC, N_PADC, 1)

  grid = N // M_TILE
  b1_2d = b1.reshape(1, D_HID)
  h_chunks = pl.pallas_call(
      _tc1_body,
      grid=(grid,),
      in_specs=[
          pl.BlockSpec((M_TILE, D_IN), lambda i: (i, 0)),
          pl.BlockSpec((NC, 1, M_TILE, D_IN), lambda i: (0, 0, i, 0)),
          pl.BlockSpec((NC, M_TILE, 1), lambda i: (0, i, 0)),
          pl.BlockSpec((D_HID, D_IN), lambda i: (0, 0)),
          pl.BlockSpec((1, D_HID), lambda i: (0, 0)),
          pl.BlockSpec((D_HID, D_IN), lambda i: (0, 0)),
      ],
      out_specs=[_chunk_spec(), _chunk_spec(), _chunk_spec(), _chunk_spec()],
      out_shape=[jax.ShapeDtypeStruct((N, 128), jnp.float32)] * 4,
  )(x, p1, pc2, W1l, b1_2d, W1r)

  p2, _ = _sc_segment_sum(src, dst, tuple(h_chunks), zf, zc,
                          with_count=False)

  b2_2d = b2.reshape(1, D_OUT)
  out = pl.pallas_call(
      _tc2_body,
      grid=(grid,),
      in_specs=[
          _chunk_spec(), _chunk_spec(), _chunk_spec(), _chunk_spec(),
          pl.BlockSpec((NC, 4, M_TILE, 128), lambda i: (0, 0, i, 0)),
          pl.BlockSpec((NC, M_TILE, 1), lambda i: (0, i, 0)),
          pl.BlockSpec((D_OUT, D_HID), lambda i: (0, 0)),
          pl.BlockSpec((1, D_OUT), lambda i: (0, 0)),
          pl.BlockSpec((D_OUT, D_HID), lambda i: (0, 0)),
      ],
      out_specs=pl.BlockSpec((M_TILE, D_OUT), lambda i: (i, 0)),
      out_shape=jax.ShapeDtypeStruct((N, D_OUT), jnp.float32),
  )(*h_chunks, p2, pc2, W2l, b2_2d, W2r)
  return out
